# baseline (speedup 1.0000x reference)
import functools

import jax
import jax.numpy as jnp
from jax import lax
from jax.experimental import pallas as pl
from jax.experimental.pallas import tpu as pltpu
from jax.experimental.pallas import tpu_sc as plsc

N_NODES = 10000
D = 128

NC = 2
NS = 16
NW = NC * NS
CHUNK = 128
N_PAD = 10240
E_PAD = 327680
N_CHUNKS = E_PAD // CHUNK
CPT = N_CHUNKS // NW
NPH = 2
CPP = CPT // NPH
NBUF = 2
ROWS_PER_TILE = N_PAD // NS
ROW_BLK = 128
DEG_PAD = 16384
DEG_PER_TILE = DEG_PAD // NS

_mesh = plsc.VectorSubcoreMesh(core_axis_name="c", subcore_axis_name="s")


def _deg_body(col_hbm, out_hbm, cidx, ones_v, zbuf, acc):
  c = lax.axis_index("c")
  s = lax.axis_index("s")
  wid = c * NS + s

  def _fill(i, _):
    zbuf[pl.ds(i * 16, 16)] = jnp.zeros((16,), jnp.float32)
    return 0
  lax.fori_loop(0, DEG_PER_TILE // 16, _fill, 0)

  def _fill1(i, _):
    ones_v[pl.ds(i * 16, 16)] = jnp.ones((16,), jnp.float32)
    return 0
  lax.fori_loop(0, CHUNK // 16, _fill1, 0)

  pltpu.sync_copy(zbuf, acc.at[pl.ds(s * DEG_PER_TILE, DEG_PER_TILE)])
  pltpu.sync_copy(col_hbm.at[pl.ds(wid * CPT, CPT)], cidx)
  plsc.subcore_barrier()

  def _body(i, _):
    pltpu.sync_copy(ones_v, acc.at[cidx.at[i]], add=True)
    return 0
  lax.fori_loop(0, CPT, _body, 0)

  plsc.subcore_barrier()
  pltpu.sync_copy(acc.at[pl.ds(s * DEG_PER_TILE, DEG_PER_TILE)], zbuf)
  pltpu.sync_copy(zbuf, out_hbm.at[c, 0, pl.ds(s * DEG_PER_TILE, DEG_PER_TILE)])


_deg_call = pl.kernel(
    _deg_body,
    out_type=jax.ShapeDtypeStruct((NC, 1, DEG_PAD), jnp.float32),
    mesh=_mesh,
    scratch_types=[
        pltpu.VMEM((CPT, CHUNK), jnp.int32),
        pltpu.VMEM((CHUNK,), jnp.float32),
        pltpu.VMEM((DEG_PER_TILE,), jnp.float32),
        pltpu.VMEM_SHARED((DEG_PAD,), jnp.float32),
    ],
)


def _scatter_body(xws_hbm, row_hbm, col_hbm, out_hbm,
                  ridx, cidx, row_v, col_v, g0, g1, acc, s0, s1):
  c = lax.axis_index("c")
  s = lax.axis_index("s")
  wid = c * NS + s

  def _zr(i, _):
    def _zc(j, _):
      g0[i, pl.ds(j * 16, 16)] = jnp.zeros((16,), jnp.float32)
      return 0
    lax.fori_loop(0, D // 16, _zc, 0)
    return 0
  lax.fori_loop(0, ROW_BLK, _zr, 0)

  def _zz(k, _):
    pltpu.sync_copy(g0, acc.at[pl.ds(s * ROWS_PER_TILE + k * ROW_BLK, ROW_BLK)])
    return 0
  lax.fori_loop(0, ROWS_PER_TILE // ROW_BLK, _zz, 0)
  plsc.subcore_barrier()

  for p in range(NPH):
    pltpu.sync_copy(row_hbm.at[pl.ds(wid * CPT + p * CPP, CPP)], ridx)
    pltpu.sync_copy(col_hbm.at[pl.ds(wid * CPT + p * CPP, CPP)], cidx)

    def _body(j, _):
      for k in range(CHUNK // 16):
        row_v[pl.ds(k * 16, 16)] = ridx[j, pl.ds(k * 16, 16)]
        col_v[pl.ds(k * 16, 16)] = cidx[j, pl.ds(k * 16, 16)]
      pltpu.async_copy(xws_hbm.at[row_v], g1, s1).wait()
      pltpu.sync_copy(g1, acc.at[col_v], add=True)
      return 0
    lax.fori_loop(0, CPP, _body, 0)

  plsc.subcore_barrier()

  def _wout(k, _):
    r0 = s * ROWS_PER_TILE + k * ROW_BLK
    pltpu.sync_copy(acc.at[pl.ds(r0, ROW_BLK)], g0)
    pltpu.sync_copy(g0, out_hbm.at[c, pl.ds(r0, ROW_BLK)])
    return 0
  lax.fori_loop(0, ROWS_PER_TILE // ROW_BLK, _wout, 0)


_scatter_call = pl.kernel(
    _scatter_body,
    out_type=jax.ShapeDtypeStruct((NC, N_PAD, D), jnp.float32),
    mesh=_mesh,
    scratch_types=[
        pltpu.VMEM((CPP, CHUNK), jnp.int32),
        pltpu.VMEM((CPP, CHUNK), jnp.int32),
        pltpu.VMEM((CHUNK,), jnp.int32),
        pltpu.VMEM((CHUNK,), jnp.int32),
        pltpu.VMEM((CHUNK, D), jnp.float32),
        pltpu.VMEM((CHUNK, D), jnp.float32),
        pltpu.VMEM_SHARED((N_PAD, D), jnp.float32),
        pltpu.SemaphoreType.DMA,
        pltpu.SemaphoreType.DMA,
    ],
)

BM = 1280
GRID = N_PAD // BM


def _tc_pre_body(deg_ref, x_ref, w_ref, xws_ref, dinv_ref):
  deg = deg_ref[0] + deg_ref[1] + 1.0
  dinv = lax.rsqrt(deg)
  dinv_ref[...] = dinv
  xw = jnp.dot(x_ref[...], w_ref[...], preferred_element_type=jnp.float32)
  xws_ref[...] = xw * dinv


_tc_pre = pl.pallas_call(
    _tc_pre_body,
    grid=(GRID,),
    in_specs=[
        pl.BlockSpec((NC, BM, 1), lambda i: (0, i, 0)),
        pl.BlockSpec((BM, D), lambda i: (i, 0)),
        pl.BlockSpec((D, D), lambda i: (0, 0)),
    ],
    out_specs=[
        pl.BlockSpec((BM, D), lambda i: (i, 0)),
        pl.BlockSpec((BM, 1), lambda i: (i, 0)),
    ],
    out_shape=[
        jax.ShapeDtypeStruct((N_PAD, D), jnp.float32),
        jax.ShapeDtypeStruct((N_PAD, 1), jnp.float32),
    ],
)


def _tc_mid_body(p0_ref, p1_ref, xws_ref, dinv_ref, b_ref, w_ref, out_ref):
  dinv = dinv_ref[...]
  h = dinv * (p0_ref[0] + p1_ref[0] + xws_ref[...]) + b_ref[...]
  h = jnp.maximum(h, 0.0)
  hw = jnp.dot(h, w_ref[...], preferred_element_type=jnp.float32)
  out_ref[...] = hw * dinv


_tc_mid = pl.pallas_call(
    _tc_mid_body,
    grid=(GRID,),
    in_specs=[
        pl.BlockSpec((1, BM, D), lambda i: (0, i, 0)),
        pl.BlockSpec((1, BM, D), lambda i: (1, i, 0)),
        pl.BlockSpec((BM, D), lambda i: (i, 0)),
        pl.BlockSpec((BM, 1), lambda i: (i, 0)),
        pl.BlockSpec((1, D), lambda i: (0, 0)),
        pl.BlockSpec((D, D), lambda i: (0, 0)),
    ],
    out_specs=pl.BlockSpec((BM, D), lambda i: (i, 0)),
    out_shape=jax.ShapeDtypeStruct((N_PAD, D), jnp.float32),
)


def _tc_post_body(p0_ref, p1_ref, xws_ref, dinv_ref, b_ref, out_ref):
  h = dinv_ref[...] * (p0_ref[0] + p1_ref[0] + xws_ref[...]) + b_ref[...]
  out_ref[...] = jnp.maximum(h, 0.0)


_tc_post = pl.pallas_call(
    _tc_post_body,
    grid=(GRID,),
    in_specs=[
        pl.BlockSpec((1, BM, D), lambda i: (0, i, 0)),
        pl.BlockSpec((1, BM, D), lambda i: (1, i, 0)),
        pl.BlockSpec((BM, D), lambda i: (i, 0)),
        pl.BlockSpec((BM, 1), lambda i: (i, 0)),
        pl.BlockSpec((1, D), lambda i: (0, 0)),
    ],
    out_specs=pl.BlockSpec((BM, D), lambda i: (i, 0)),
    out_shape=jax.ShapeDtypeStruct((N_PAD, D), jnp.float32),
)


@jax.jit
def kernel(x, edge_index, W1, b1, W2, b2):
  ei = edge_index.astype(jnp.int32)
  pad = jnp.full((2, E_PAD - ei.shape[1]), N_PAD - 1, jnp.int32)
  ei = jnp.concatenate([ei, pad], axis=1)
  row3 = ei[0].reshape(N_CHUNKS, CHUNK)
  col3 = ei[1].reshape(N_CHUNKS, CHUNK)
  x_pad = jnp.zeros((N_PAD, D), x.dtype).at[:N_NODES].set(x)
  b1r = b1.reshape(1, D)
  b2r = b2.reshape(1, D)

  deg_p = _deg_call(col3)
  deg3 = deg_p[:, 0, :N_PAD, None]
  xws1, dinv = _tc_pre(deg3, x_pad, W1)
  acc1 = _scatter_call(xws1, row3, col3)
  xws2 = _tc_mid(acc1, acc1, xws1, dinv, b1r, W2)
  acc2 = _scatter_call(xws2, row3, col3)
  return _tc_post(acc2, acc2, xws2, dinv, b2r)[:N_NODES]


# speedup vs baseline: 2.0061x; 2.0061x over previous
import functools

import jax
import jax.numpy as jnp
from jax import lax
from jax.experimental import pallas as pl
from jax.experimental.pallas import tpu as pltpu
from jax.experimental.pallas import tpu_sc as plsc

N_NODES = 10000
N_EDGES = 320000
D = 128

NC = 2
NS = 16
NW = NC * NS
CHUNK = 128
N_CHUNKS = N_EDGES // CHUNK
N_PAD = 10240
ROWS_PER_TILE = N_PAD // NS
ROW_BLK = 128
DEG_PAD = 16384
DEG_PER_TILE = DEG_PAD // NS

_mesh = plsc.VectorSubcoreMesh(core_axis_name="c", subcore_axis_name="s")


def _deg_body(col_hbm, out_hbm, col_v, ones_v, zbuf, acc):
  c = lax.axis_index("c")
  s = lax.axis_index("s")
  wid = c * NS + s

  def _fill(i, _):
    zbuf[pl.ds(i * 16, 16)] = jnp.zeros((16,), jnp.float32)
    return 0
  lax.fori_loop(0, DEG_PER_TILE // 16, _fill, 0)

  def _fill1(i, _):
    ones_v[pl.ds(i * 16, 16)] = jnp.ones((16,), jnp.float32)
    return 0
  lax.fori_loop(0, CHUNK // 16, _fill1, 0)

  pltpu.sync_copy(zbuf, acc.at[pl.ds(s * DEG_PER_TILE, DEG_PER_TILE)])
  plsc.subcore_barrier()

  nmine = jnp.where(wid < N_CHUNKS % NW, N_CHUNKS // NW + 1, N_CHUNKS // NW)

  def _body(i, _):
    base = (wid + i * NW) * CHUNK
    pltpu.sync_copy(col_hbm.at[pl.ds(base, CHUNK)], col_v)
    pltpu.sync_copy(ones_v, acc.at[col_v], add=True)
    return 0
  lax.fori_loop(0, nmine, _body, 0)

  plsc.subcore_barrier()
  pltpu.sync_copy(acc.at[pl.ds(s * DEG_PER_TILE, DEG_PER_TILE)], zbuf)
  pltpu.sync_copy(zbuf, out_hbm.at[c, 0, pl.ds(s * DEG_PER_TILE, DEG_PER_TILE)])


_deg_call = pl.kernel(
    _deg_body,
    out_type=jax.ShapeDtypeStruct((NC, 1, DEG_PAD), jnp.float32),
    mesh=_mesh,
    scratch_types=[
        pltpu.VMEM((CHUNK,), jnp.int32),
        pltpu.VMEM((CHUNK,), jnp.float32),
        pltpu.VMEM((DEG_PER_TILE,), jnp.float32),
        pltpu.VMEM_SHARED((DEG_PAD,), jnp.float32),
    ],
)


def _scatter_body(xws_hbm, row_hbm, col_hbm, out_hbm,
                  row_v0, col_v0, row_v1, col_v1, g0, g1, acc,
                  sg0, sg1, sa, sb):
  c = lax.axis_index("c")
  s = lax.axis_index("s")
  wid = c * NS + s

  def _zr(i, _):
    def _zc(j, _):
      g0[i, pl.ds(j * 16, 16)] = jnp.zeros((16,), jnp.float32)
      return 0
    lax.fori_loop(0, D // 16, _zc, 0)
    return 0
  lax.fori_loop(0, ROW_BLK, _zr, 0)

  def _zz(k, _):
    pltpu.sync_copy(g0, acc.at[pl.ds(s * ROWS_PER_TILE + k * ROW_BLK, ROW_BLK)])
    return 0
  lax.fori_loop(0, ROWS_PER_TILE // ROW_BLK, _zz, 0)
  plsc.subcore_barrier()

  def _body(i, _):
    b0 = (wid + (2 * i) * NW) * CHUNK
    b1 = (wid + (2 * i + 1) * NW) * CHUNK
    pltpu.sync_copy(row_hbm.at[pl.ds(b0, CHUNK)], row_v0)
    pltpu.sync_copy(col_hbm.at[pl.ds(b0, CHUNK)], col_v0)
    pltpu.async_copy(xws_hbm.at[row_v0], g0, sg0).wait()
    da = pltpu.async_copy(g0, acc.at[col_v0], sa, add=True)
    pltpu.sync_copy(row_hbm.at[pl.ds(b1, CHUNK)], row_v1)
    pltpu.sync_copy(col_hbm.at[pl.ds(b1, CHUNK)], col_v1)
    pltpu.async_copy(xws_hbm.at[row_v1], g1, sg1).wait()
    da.wait()
    db = pltpu.async_copy(g1, acc.at[col_v1], sb, add=True)
    db.wait()
    return 0
  lax.fori_loop(0, (N_CHUNKS // NW) // 2, _body, 0)

  @pl.when(wid < N_CHUNKS % NW)
  def _tail():
    base = (wid + (N_CHUNKS // NW) * NW) * CHUNK
    pltpu.sync_copy(row_hbm.at[pl.ds(base, CHUNK)], row_v0)
    pltpu.sync_copy(col_hbm.at[pl.ds(base, CHUNK)], col_v0)
    pltpu.async_copy(xws_hbm.at[row_v0], g0, sg0).wait()
    pltpu.sync_copy(g0, acc.at[col_v0], add=True)

  plsc.subcore_barrier()

  def _wout(k, _):
    r0 = s * ROWS_PER_TILE + k * ROW_BLK
    pltpu.sync_copy(acc.at[pl.ds(r0, ROW_BLK)], g0)
    pltpu.sync_copy(g0, out_hbm.at[c, pl.ds(r0, ROW_BLK)])
    return 0
  lax.fori_loop(0, ROWS_PER_TILE // ROW_BLK, _wout, 0)


_scatter_call = pl.kernel(
    _scatter_body,
    out_type=jax.ShapeDtypeStruct((NC, N_PAD, D), jnp.float32),
    mesh=_mesh,
    scratch_types=[
        pltpu.VMEM((CHUNK,), jnp.int32),
        pltpu.VMEM((CHUNK,), jnp.int32),
        pltpu.VMEM((CHUNK,), jnp.int32),
        pltpu.VMEM((CHUNK,), jnp.int32),
        pltpu.VMEM((CHUNK, D), jnp.float32),
        pltpu.VMEM((CHUNK, D), jnp.float32),
        pltpu.VMEM_SHARED((N_PAD, D), jnp.float32),
        pltpu.SemaphoreType.DMA,
        pltpu.SemaphoreType.DMA,
        pltpu.SemaphoreType.DMA,
        pltpu.SemaphoreType.DMA,
    ],
)

BM = 2000
GRID = N_NODES // BM


def _tc_pre_body(deg_ref, x_ref, w_ref, xws_ref, dinv_ref):
  deg = deg_ref[0] + deg_ref[1] + 1.0
  dinv = lax.rsqrt(deg)
  dinv_ref[...] = dinv
  xw = jnp.dot(x_ref[...], w_ref[...], preferred_element_type=jnp.float32)
  xws_ref[...] = xw * dinv


_tc_pre = pl.pallas_call(
    _tc_pre_body,
    grid=(GRID,),
    in_specs=[
        pl.BlockSpec((NC, BM, 1), lambda i: (0, i, 0)),
        pl.BlockSpec((BM, D), lambda i: (i, 0)),
        pl.BlockSpec((D, D), lambda i: (0, 0)),
    ],
    out_specs=[
        pl.BlockSpec((BM, D), lambda i: (i, 0)),
        pl.BlockSpec((BM, 1), lambda i: (i, 0)),
    ],
    out_shape=[
        jax.ShapeDtypeStruct((N_NODES, D), jnp.float32),
        jax.ShapeDtypeStruct((N_NODES, 1), jnp.float32),
    ],
)


def _tc_mid_body(p0_ref, p1_ref, xws_ref, dinv_ref, b_ref, w_ref, out_ref):
  dinv = dinv_ref[...]
  h = dinv * (p0_ref[0] + p1_ref[0] + xws_ref[...]) + b_ref[...]
  h = jnp.maximum(h, 0.0)
  hw = jnp.dot(h, w_ref[...], preferred_element_type=jnp.float32)
  out_ref[...] = hw * dinv


_tc_mid = pl.pallas_call(
    _tc_mid_body,
    grid=(GRID,),
    in_specs=[
        pl.BlockSpec((1, BM, D), lambda i: (0, i, 0)),
        pl.BlockSpec((1, BM, D), lambda i: (1, i, 0)),
        pl.BlockSpec((BM, D), lambda i: (i, 0)),
        pl.BlockSpec((BM, 1), lambda i: (i, 0)),
        pl.BlockSpec((1, D), lambda i: (0, 0)),
        pl.BlockSpec((D, D), lambda i: (0, 0)),
    ],
    out_specs=pl.BlockSpec((BM, D), lambda i: (i, 0)),
    out_shape=jax.ShapeDtypeStruct((N_NODES, D), jnp.float32),
)


def _tc_post_body(p0_ref, p1_ref, xws_ref, dinv_ref, b_ref, out_ref):
  h = dinv_ref[...] * (p0_ref[0] + p1_ref[0] + xws_ref[...]) + b_ref[...]
  out_ref[...] = jnp.maximum(h, 0.0)


_tc_post = pl.pallas_call(
    _tc_post_body,
    grid=(GRID,),
    in_specs=[
        pl.BlockSpec((1, BM, D), lambda i: (0, i, 0)),
        pl.BlockSpec((1, BM, D), lambda i: (1, i, 0)),
        pl.BlockSpec((BM, D), lambda i: (i, 0)),
        pl.BlockSpec((BM, 1), lambda i: (i, 0)),
        pl.BlockSpec((1, D), lambda i: (0, 0)),
    ],
    out_specs=pl.BlockSpec((BM, D), lambda i: (i, 0)),
    out_shape=jax.ShapeDtypeStruct((N_NODES, D), jnp.float32),
)


@jax.jit
def kernel(x, edge_index, W1, b1, W2, b2):
  ei = edge_index.astype(jnp.int32)
  row = ei[0]
  col = ei[1]
  b1r = b1.reshape(1, D)
  b2r = b2.reshape(1, D)

  deg_p = _deg_call(col)
  deg3 = deg_p[:, 0, :N_NODES, None]
  xws1, dinv = _tc_pre(deg3, x, W1)
  acc1 = _scatter_call(xws1, row, col)
  xws2 = _tc_mid(acc1, acc1, xws1, dinv, b1r, W2)
  acc2 = _scatter_call(xws2, row, col)
  return _tc_post(acc2, acc2, xws2, dinv, b2r)
